# Initial kernel scaffold; baseline (speedup 1.0000x reference)
import jax
import jax.numpy as jnp
from jax.experimental import pallas as pl


def kernel(img_batch, Mask, rand_category, rand_index):
    raise NotImplementedError("write your pallas kernel here")


# speedup vs baseline: 3.0820x; 3.0820x over previous
import jax
import jax.numpy as jnp
from jax.experimental import pallas as pl
from jax.experimental.pallas import tpu as pltpu

_IMGSIZE = 640
_MASKNUM = 10
_POS0 = 0.001
_POS1 = 0.5


def _blend_body(idx_ref, img_ref, mask_ref, out_ref):
    m = mask_ref[...]
    out_ref[...] = img_ref[...] * (1.0 - m) + m


def kernel(img_batch, Mask, rand_category, rand_index):
    B, C, H, W = img_batch.shape
    mask_num = Mask.shape[0]

    category = jnp.where(rand_category <= _POS0, 0.0, 1.0)
    category = jnp.where(rand_category > _POS1, 2.0, category)
    fidx = (category - 1.0) * _MASKNUM + rand_index * _MASKNUM
    idx = jnp.ceil(fidx).astype(jnp.int32)
    idx = jnp.clip(idx, 0, mask_num - 1)

    grid_spec = pltpu.PrefetchScalarGridSpec(
        num_scalar_prefetch=1,
        grid=(B, C),
        in_specs=[
            pl.BlockSpec((1, 1, H, W), lambda b, c, idx_ref: (b, c, 0, 0)),
            pl.BlockSpec((1, 1, H, W), lambda b, c, idx_ref: (idx_ref[b], c, 0, 0)),
        ],
        out_specs=pl.BlockSpec((1, 1, H, W), lambda b, c, idx_ref: (b, c, 0, 0)),
    )
    return pl.pallas_call(
        _blend_body,
        grid_spec=grid_spec,
        out_shape=jax.ShapeDtypeStruct((B, C, H, W), img_batch.dtype),
    )(idx, img_batch, Mask)
